# baseline (speedup 1.0000x reference)
import functools

import jax
import jax.numpy as jnp
from jax import lax
from jax.experimental import pallas as pl
from jax.experimental.pallas import tpu as pltpu
from jax.experimental.pallas import tpu_sc as plsc

_NC = 2
_NS = 16
_NW = _NC * _NS
_CH = 128


def _sc_embed(idx3, table):
    n_ch = idx3.shape[1]
    d = table.shape[1]
    mesh = plsc.VectorSubcoreMesh(core_axis_name="c", subcore_axis_name="s")

    @functools.partial(
        pl.kernel,
        mesh=mesh,
        out_type=jax.ShapeDtypeStruct((_NW, n_ch, _CH, d), jnp.float32),
        scratch_types=[
            pltpu.VMEM((n_ch, _CH), jnp.int32),
            pltpu.VMEM((_CH, d), jnp.float32),
            pltpu.SemaphoreType.DMA,
        ],
        compiler_params=pltpu.CompilerParams(use_tc_tiling_on_sc=False),
    )
    def k(idx_hbm, table_hbm, out_hbm, idx_v, rows_v, sem):
        wid = lax.axis_index("s") * _NC + lax.axis_index("c")
        pltpu.sync_copy(idx_hbm.at[wid], idx_v)

        def body(j, carry):
            pltpu.async_copy(table_hbm.at[idx_v.at[j]], rows_v, sem).wait()
            pltpu.sync_copy(rows_v, out_hbm.at[wid, j])
            return carry

        lax.fori_loop(0, n_ch, body, 0)

    return k(idx3, table)


def kernel(x, table):
    s, t = x.shape
    d = table.shape[1]
    b = s * t
    n_ch = b // (_NW * _CH)
    idx3 = x.reshape(_NW, n_ch, _CH).astype(jnp.int32)
    out = _sc_embed(idx3, table)
    return out.reshape(s, t, d)


# speedup vs baseline: 1.1150x; 1.1150x over previous
import functools

import jax
import jax.numpy as jnp
from jax import lax
from jax.experimental import pallas as pl
from jax.experimental.pallas import tpu as pltpu
from jax.experimental.pallas import tpu_sc as plsc

_NC = 2
_NS = 16
_NW = _NC * _NS
_CH = 128
_NBUF = 8
_LEAD = 4


def _sc_embed(idx3, table):
    n_ch = idx3.shape[1]
    d = table.shape[1]
    mesh = plsc.VectorSubcoreMesh(core_axis_name="c", subcore_axis_name="s")

    @functools.partial(
        pl.kernel,
        mesh=mesh,
        out_type=jax.ShapeDtypeStruct((_NW, n_ch, _CH, d), jnp.float32),
        scratch_types=[
            pltpu.VMEM((n_ch, _CH), jnp.int32),
            pltpu.VMEM((_NBUF, _CH, d), jnp.float32),
            pltpu.SemaphoreType.DMA((_NBUF,)),
            pltpu.SemaphoreType.DMA((_NBUF,)),
        ],
        compiler_params=pltpu.CompilerParams(use_tc_tiling_on_sc=False),
    )
    def k(idx_hbm, table_hbm, out_hbm, idx_v, rows_v, sem_g, sem_w):
        wid = lax.axis_index("s") * _NC + lax.axis_index("c")
        pltpu.sync_copy(idx_hbm.at[wid], idx_v)

        def start_gather(j, b):
            pltpu.async_copy(table_hbm.at[idx_v.at[j]], rows_v.at[b], sem_g.at[b])

        def wait_gather(j, b):
            pltpu.make_async_copy(
                table_hbm.at[idx_v.at[j]], rows_v.at[b], sem_g.at[b]).wait()

        def start_write(j, b):
            pltpu.async_copy(rows_v.at[b], out_hbm.at[wid, j], sem_w.at[b])

        def wait_write(j, b):
            pltpu.make_async_copy(
                rows_v.at[b], out_hbm.at[wid, j], sem_w.at[b]).wait()

        for b in range(_LEAD):
            start_gather(b, b)
        for j in range(_LEAD):
            start_gather(j + _LEAD, j + _LEAD)
            wait_gather(j, j)
            start_write(j, j)

        n_steady = n_ch - 2 * _LEAD
        n_groups = n_steady // _NBUF

        def body(q, carry):
            jb = q * _NBUF + _LEAD
            for r in range(_NBUF):
                j = jb + r
                b = (_LEAD + r) % _NBUF
                bn = r
                wait_write(j - _LEAD, bn)
                start_gather(j + _LEAD, bn)
                wait_gather(j, b)
                start_write(j, b)
            return carry

        lax.fori_loop(0, n_groups, body, 0)

        for r in range(_LEAD):
            j = n_ch - _LEAD + r
            b = j % _NBUF
            bn = (j + _LEAD) % _NBUF
            wait_write(j - _LEAD, bn)
            wait_gather(j, b)
            start_write(j, b)
        for r in range(_LEAD):
            j = n_ch - _LEAD + r
            wait_write(j, j % _NBUF)

    return k(idx3, table)


def kernel(x, table):
    s, t = x.shape
    d = table.shape[1]
    b = s * t
    n_ch = b // (_NW * _CH)
    idx3 = x.reshape(_NW, n_ch, _CH).astype(jnp.int32)
    out = _sc_embed(idx3, table)
    return out.reshape(s, t, d)


# speedup vs baseline: 1.3652x; 1.2244x over previous
import functools

import jax
import jax.numpy as jnp
from jax import lax
from jax.experimental import pallas as pl
from jax.experimental.pallas import tpu as pltpu
from jax.experimental.pallas import tpu_sc as plsc

_NC = 2
_NS = 16
_NW = _NC * _NS
_CH = 128
_NBUF = 4
_LEAD = 2


def _sc_embed(idx3, tpad, d):
    n_ch = idx3.shape[1]
    dp = tpad.shape[1]
    b = _NW * n_ch * _CH
    mesh = plsc.VectorSubcoreMesh(core_axis_name="c", subcore_axis_name="s")

    @functools.partial(
        pl.kernel,
        mesh=mesh,
        out_type=jax.ShapeDtypeStruct((b, dp), jnp.float32),
        scratch_types=[
            pltpu.VMEM((n_ch, _CH), jnp.int32),
            pltpu.VMEM((_NBUF, _CH, dp), jnp.float32),
            pltpu.SemaphoreType.DMA((_NBUF,)),
            pltpu.SemaphoreType.DMA((_NBUF,)),
        ],
    )
    def k(idx_hbm, table_hbm, out_hbm, idx_v, rows_v, sem_g, sem_w):
        wid = lax.axis_index("s") * _NC + lax.axis_index("c")
        base = wid * n_ch * _CH
        pltpu.sync_copy(idx_hbm.at[wid], idx_v)

        def start_gather(j, bu):
            pltpu.async_copy(table_hbm.at[idx_v.at[j]], rows_v.at[bu], sem_g.at[bu])

        def wait_gather(j, bu):
            pltpu.make_async_copy(
                table_hbm.at[idx_v.at[j]], rows_v.at[bu], sem_g.at[bu]).wait()

        def start_write(j, bu):
            pltpu.async_copy(
                rows_v.at[bu], out_hbm.at[pl.ds(base + j * _CH, _CH)],
                sem_w.at[bu])

        def wait_write(j, bu):
            pltpu.make_async_copy(
                rows_v.at[bu], out_hbm.at[pl.ds(base + j * _CH, _CH)],
                sem_w.at[bu]).wait()

        for bu in range(_LEAD):
            start_gather(bu, bu)
        for j in range(_LEAD):
            start_gather(j + _LEAD, j + _LEAD)
            wait_gather(j, j)
            start_write(j, j)

        n_steady = n_ch - 2 * _LEAD
        n_groups = n_steady // _NBUF

        def body(q, carry):
            jb = q * _NBUF + _LEAD
            for r in range(_NBUF):
                j = jb + r
                bu = (_LEAD + r) % _NBUF
                bn = r
                wait_write(j - _LEAD, bn)
                start_gather(j + _LEAD, bn)
                wait_gather(j, bu)
                start_write(j, bu)
            return carry

        lax.fori_loop(0, n_groups, body, 0)

        for r in range(_LEAD):
            j = n_ch - _LEAD + r
            bu = j % _NBUF
            bn = (j + _LEAD) % _NBUF
            wait_write(j - _LEAD, bn)
            wait_gather(j, bu)
            start_write(j, bu)
        for r in range(_LEAD):
            j = n_ch - _LEAD + r
            wait_write(j, j % _NBUF)

    return k(idx3, tpad)


def kernel(x, table):
    s, t = x.shape
    v, d = table.shape
    b = s * t
    n_ch = b // (_NW * _CH)
    tpad = jnp.pad(table, ((0, 0), (0, 128 - d)))
    idx3 = x.reshape(_NW, n_ch, _CH).astype(jnp.int32)
    out = _sc_embed(idx3, tpad, d)
    return out[:, :d].reshape(s, t, d)
